# baseline (speedup 1.0000x reference)
import functools

import jax
import jax.numpy as jnp
from jax import lax
from jax.experimental import pallas as pl
from jax.experimental.pallas import tpu as pltpu
from jax.experimental.pallas import tpu_sc as plsc

_NUM_CORES = 2
_NUM_SUBCORES = 16
_NW = _NUM_CORES * _NUM_SUBCORES
_LANES = 16
_CHUNK = 128
_BS = 2048
_TC_FRAC = 2


@functools.cache
def _gather_build(B, D):
    R = B // _TC_FRAC
    rpw = R // _NW
    nsub = rpw // _CHUNK
    mesh = plsc.VectorSubcoreMesh(core_axis_name="c", subcore_axis_name="s")

    @functools.partial(
        pl.kernel,
        out_type=jax.ShapeDtypeStruct((R, D), jnp.float32),
        mesh=mesh,
        scratch_types=[
            pltpu.VMEM((nsub, _CHUNK), jnp.int32),
            pltpu.VMEM((nsub, _CHUNK, D), jnp.float32),
            [pltpu.SemaphoreType.DMA] * 2,
            pltpu.SemaphoreType.DMA,
        ],
    )
    def gk(label_hbm, centers_hbm, out_hbm, idx_v, gbuf_v, gsems, wsem):
        wid = lax.axis_index("s") * _NUM_CORES + lax.axis_index("c")
        pltpu.sync_copy(label_hbm.at[pl.ds(wid * nsub, nsub), :], idx_v)
        gds = [
            pltpu.async_copy(centers_hbm.at[idx_v.at[s]],
                             gbuf_v.at[s], gsems[s % 2])
            for s in range(nsub)
        ]
        wds = []
        for s in range(nsub):
            gds[s].wait()
            wds.append(pltpu.async_copy(
                gbuf_v.at[s],
                out_hbm.at[pl.ds(wid * rpw + s * _CHUNK, _CHUNK), :],
                wsem))
        for w in wds:
            w.wait()

    return gk


@functools.cache
def _compute_build(B, D):
    row_lo = B // _TC_FRAC
    rows_per_w = (B - row_lo) // _NW
    n_chunks = rows_per_w // _CHUNK
    vecs_per_row = D // _LANES
    mesh = plsc.VectorSubcoreMesh(core_axis_name="c", subcore_axis_name="s")

    @functools.partial(
        pl.kernel,
        out_type=jax.ShapeDtypeStruct((_NW * _LANES,), jnp.float32),
        mesh=mesh,
        scratch_types=[
            pltpu.VMEM((n_chunks, _CHUNK), jnp.int32),
            pltpu.VMEM((2, _CHUNK, D), jnp.float32),
            pltpu.VMEM((2, _CHUNK, D), jnp.float32),
            pltpu.VMEM((2, _CHUNK, D), jnp.float32),
            pltpu.VMEM((_LANES,), jnp.float32),
            [pltpu.SemaphoreType.DMA] * 6,
        ],
    )
    def sc_kernel(feat_hbm, a_hbm, label_hbm, centers_hbm, out_hbm,
                  idx_v, cent_v, feat_v, a_v, acc_v, sems):
        wid = lax.axis_index("s") * _NUM_CORES + lax.axis_index("c")
        base = row_lo + wid * rows_per_w
        lbl0 = row_lo // _CHUNK + wid * n_chunks

        def fire_linear(ci, slot):
            row0 = base + ci * _CHUNK
            return (
                pltpu.async_copy(feat_hbm.at[pl.ds(row0, _CHUNK), :],
                                 feat_v.at[slot], sems[3 * slot + 1]),
                pltpu.async_copy(a_hbm.at[pl.ds(row0, _CHUNK), :],
                                 a_v.at[slot], sems[3 * slot + 2]),
            )

        def fire_gather(ci, slot):
            return pltpu.async_copy(centers_hbm.at[idx_v.at[ci]],
                                    cent_v.at[slot], sems[3 * slot])

        lin0 = fire_linear(0, 0)
        pltpu.sync_copy(label_hbm.at[pl.ds(lbl0, n_chunks), :], idx_v)
        in_flight = lin0 + (fire_gather(0, 0),)

        acc = tuple(jnp.zeros((_LANES,), jnp.float32)
                    for _ in range(vecs_per_row))
        for ci in range(n_chunks):
            slot = ci % 2
            cur = in_flight
            if ci + 1 < n_chunks:
                in_flight = ((fire_gather(ci + 1, 1 - slot),)
                             + fire_linear(ci + 1, 1 - slot))
            for cp in cur:
                cp.wait()

            def row_body(r, accs):
                new = []
                for j in range(vecs_per_row):
                    f = feat_v[slot, r, pl.ds(j * _LANES, _LANES)]
                    c = cent_v[slot, r, pl.ds(j * _LANES, _LANES)]
                    w = a_v[slot, r, pl.ds(j * _LANES, _LANES)]
                    d = f - c
                    new.append(accs[j] + w * d * d)
                return tuple(new)

            acc = lax.fori_loop(0, _CHUNK, row_body, acc)
        total = acc[0]
        for j in range(1, vecs_per_row):
            total = total + acc[j]
        acc_v[...] = total
        pltpu.sync_copy(acc_v, out_hbm.at[pl.ds(wid * _LANES, _LANES)])

    return sc_kernel


@functools.cache
def _reduce_build(B, D):
    R = B // _TC_FRAC
    grid = R // _BS

    def rk(feat_ref, a_ref, g_ref, o_ref):
        d = feat_ref[...] - g_ref[...]
        t = a_ref[...] * d * d
        o_ref[...] = jnp.broadcast_to(
            jnp.sum(t, axis=0, keepdims=True), (8, D))

    return pl.pallas_call(
        rk,
        grid=(grid,),
        in_specs=[
            pl.BlockSpec((_BS, D), lambda i: (i, 0)),
            pl.BlockSpec((_BS, D), lambda i: (i, 0)),
            pl.BlockSpec((_BS, D), lambda i: (i, 0)),
        ],
        out_specs=pl.BlockSpec((8, D), lambda i: (i, 0)),
        out_shape=jax.ShapeDtypeStruct((grid * 8, D), jnp.float32),
    )


def kernel(feat, A, label, centers):
    B, D = feat.shape
    label2d = label.astype(jnp.int32).reshape(B // _CHUNK, _CHUNK)
    g = _gather_build(B, D)(label2d, centers)
    sc_part = _compute_build(B, D)(feat, A, label2d, centers)
    tc_part = _reduce_build(B, D)(feat, A, g)
    total = jnp.sum(sc_part) + jnp.sum(tc_part) * (1.0 / 8.0)
    return total * (0.5 / B)


# speedup vs baseline: 1.0584x; 1.0584x over previous
import functools

import jax
import jax.numpy as jnp
from jax import lax
from jax.experimental import pallas as pl
from jax.experimental.pallas import tpu as pltpu
from jax.experimental.pallas import tpu_sc as plsc

_NUM_CORES = 2
_NUM_SUBCORES = 16
_NW = _NUM_CORES * _NUM_SUBCORES
_LANES = 16
_IDXW = 64
_CHUNKS = (128, 128, 128, 64, 64)
_BUFROWS = 128


@functools.cache
def _build(B, D):
    rows_per_w = B // _NW
    vecs_per_row = D // _LANES
    assert sum(_CHUNKS) == rows_per_w
    n_chunks = len(_CHUNKS)
    starts = [sum(_CHUNKS[:i]) for i in range(n_chunks)]
    idx_rows_per_w = rows_per_w // _IDXW

    mesh = plsc.VectorSubcoreMesh(core_axis_name="c", subcore_axis_name="s")

    @functools.partial(
        pl.kernel,
        out_type=jax.ShapeDtypeStruct((_NW * _LANES,), jnp.float32),
        mesh=mesh,
        scratch_types=[
            pltpu.VMEM((idx_rows_per_w, _IDXW), jnp.int32),
            pltpu.VMEM((2, _BUFROWS, D), jnp.float32),
            pltpu.VMEM((2, _BUFROWS, D), jnp.float32),
            pltpu.VMEM((2, _BUFROWS, D), jnp.float32),
            pltpu.VMEM((_LANES,), jnp.float32),
            [pltpu.SemaphoreType.DMA] * 6,
        ],
    )
    def sc_kernel(feat_hbm, a_hbm, label_hbm, centers_hbm, out_hbm,
                  idx_v, cent_v, feat_v, a_v, acc_v, sems):
        wid = lax.axis_index("s") * _NUM_CORES + lax.axis_index("c")
        base = wid * rows_per_w

        def fire_linear(ci, slot):
            n = _CHUNKS[ci]
            row0 = base + starts[ci]
            return (
                pltpu.async_copy(feat_hbm.at[pl.ds(row0, n), :],
                                 feat_v.at[slot, pl.ds(0, n), :],
                                 sems[3 * slot + 1]),
                pltpu.async_copy(a_hbm.at[pl.ds(row0, n), :],
                                 a_v.at[slot, pl.ds(0, n), :],
                                 sems[3 * slot + 2]),
            )

        def fire_gather(ci, slot):
            n = _CHUNKS[ci]
            r64 = starts[ci] // _IDXW
            return tuple(
                pltpu.async_copy(
                    centers_hbm.at[idx_v.at[r64 + k]],
                    cent_v.at[slot, pl.ds(k * _IDXW, _IDXW), :],
                    sems[3 * slot])
                for k in range(n // _IDXW)
            )

        lin0 = fire_linear(0, 0)
        pltpu.sync_copy(
            label_hbm.at[pl.ds(wid * idx_rows_per_w, idx_rows_per_w), :],
            idx_v)
        in_flight = lin0 + fire_gather(0, 0)

        acc = tuple(jnp.zeros((_LANES,), jnp.float32)
                    for _ in range(vecs_per_row))
        for ci in range(n_chunks):
            slot = ci % 2
            cur = in_flight
            if ci + 1 < n_chunks:
                in_flight = (fire_gather(ci + 1, 1 - slot)
                             + fire_linear(ci + 1, 1 - slot))
            for cp in cur:
                cp.wait()

            def row_body(r, accs):
                new = []
                for j in range(vecs_per_row):
                    f = feat_v[slot, r, pl.ds(j * _LANES, _LANES)]
                    c = cent_v[slot, r, pl.ds(j * _LANES, _LANES)]
                    w = a_v[slot, r, pl.ds(j * _LANES, _LANES)]
                    d = f - c
                    new.append(accs[j] + w * d * d)
                return tuple(new)

            acc = lax.fori_loop(0, _CHUNKS[ci], row_body, acc)
        total = acc[0]
        for j in range(1, vecs_per_row):
            total = total + acc[j]
        acc_v[...] = total
        pltpu.sync_copy(acc_v, out_hbm.at[pl.ds(wid * _LANES, _LANES)])

    return sc_kernel


def kernel(feat, A, label, centers):
    B, D = feat.shape
    label2d = label.astype(jnp.int32).reshape(B // _IDXW, _IDXW)
    partials = _build(B, D)(feat, A, label2d, centers)
    return jnp.sum(partials) * (0.5 / B)


# speedup vs baseline: 1.0957x; 1.0352x over previous
import functools

import jax
import jax.numpy as jnp
from jax import lax
from jax.experimental import pallas as pl
from jax.experimental.pallas import tpu as pltpu
from jax.experimental.pallas import tpu_sc as plsc

_NUM_CORES = 2
_NUM_SUBCORES = 16
_NW = _NUM_CORES * _NUM_SUBCORES
_LANES = 16
_CHUNK = 128


@functools.cache
def _build(B, D):
    rows_per_w = B // _NW
    n_chunks = rows_per_w // _CHUNK
    vecs_per_row = D // _LANES
    assert rows_per_w * _NW == B and n_chunks * _CHUNK == rows_per_w
    assert vecs_per_row * _LANES == D

    mesh = plsc.VectorSubcoreMesh(core_axis_name="c", subcore_axis_name="s")

    @functools.partial(
        pl.kernel,
        out_type=jax.ShapeDtypeStruct((_NW * _LANES,), jnp.float32),
        mesh=mesh,
        scratch_types=[
            pltpu.VMEM((n_chunks, _CHUNK), jnp.int32),
            pltpu.VMEM((2, _CHUNK, D), jnp.float32),
            pltpu.VMEM((2, _CHUNK, D), jnp.float32),
            pltpu.VMEM((2, _CHUNK, D), jnp.float32),
            pltpu.VMEM((_LANES,), jnp.float32),
            [pltpu.SemaphoreType.DMA] * 6,
        ],
    )
    def sc_kernel(feat_hbm, a_hbm, label_hbm, centers_hbm, out_hbm,
                  idx_v, cent_v, feat_v, a_v, acc_v, sems):
        wid = lax.axis_index("s") * _NUM_CORES + lax.axis_index("c")
        base = wid * rows_per_w

        def fire_linear(ci, slot):
            row0 = base + ci * _CHUNK
            return (
                pltpu.async_copy(feat_hbm.at[pl.ds(row0, _CHUNK), :],
                                 feat_v.at[slot], sems[3 * slot + 1]),
                pltpu.async_copy(a_hbm.at[pl.ds(row0, _CHUNK), :],
                                 a_v.at[slot], sems[3 * slot + 2]),
            )

        def fire_gather(ci, slot):
            return pltpu.async_copy(centers_hbm.at[idx_v.at[ci]],
                                    cent_v.at[slot], sems[3 * slot])

        lin0 = fire_linear(0, 0)
        pltpu.sync_copy(label_hbm.at[pl.ds(wid * n_chunks, n_chunks), :],
                        idx_v)
        in_flight = lin0 + (fire_gather(0, 0),)

        acc = tuple(jnp.zeros((_LANES,), jnp.float32)
                    for _ in range(vecs_per_row))
        for ci in range(n_chunks):
            slot = ci % 2
            cur = in_flight
            if ci + 1 < n_chunks:
                in_flight = ((fire_gather(ci + 1, 1 - slot),)
                             + fire_linear(ci + 1, 1 - slot))
            for cp in cur:
                cp.wait()

            def row_body(r, accs):
                new = []
                for j in range(vecs_per_row):
                    f = feat_v[slot, r, pl.ds(j * _LANES, _LANES)]
                    c = cent_v[slot, r, pl.ds(j * _LANES, _LANES)]
                    w = a_v[slot, r, pl.ds(j * _LANES, _LANES)]
                    d = f - c
                    new.append(accs[j] + w * d * d)
                return tuple(new)

            acc = lax.fori_loop(0, _CHUNK, row_body, acc)
        total = acc[0]
        for j in range(1, vecs_per_row):
            total = total + acc[j]
        acc_v[...] = total
        pltpu.sync_copy(acc_v, out_hbm.at[pl.ds(wid * _LANES, _LANES)])

    return sc_kernel


def kernel(feat, A, label, centers):
    B, D = feat.shape
    label2d = label.astype(jnp.int32).reshape(B // _CHUNK, _CHUNK)
    partials = _build(B, D)(feat, A, label2d, centers)
    return jnp.sum(partials) * (0.5 / B)
